# Initial kernel scaffold; baseline (speedup 1.0000x reference)
import functools

import jax
import jax.numpy as jnp
from jax import lax
from jax.experimental import pallas as pl
from jax.experimental.pallas import tpu as pltpu
from jax.experimental.pallas import tpu_sc as plsc

N_NODES = 10000
N_EDGES = 320000
D_FEAT = 128

NC = 2
NS = 16
NW = NC * NS
B_EDGE = 80
EPW = N_EDGES // NW
TPW = EPW // B_EDGE
ROWS_PER_TEC = N_NODES // NS
STAGE_ROWS = 125
LANES = 16
CPD = D_FEAT // LANES


def _sc_body(x_hbm, wij_hbm, idxi_hbm, idxj_hbm, out_hbm,
             idxi_v, idxj_v, wij_v, xr_v, stage_v, y_sh, sem):
    c = lax.axis_index("c")
    s = lax.axis_index("s")
    wid = s * NC + c

    def _zero_row(r, _):
        for cc in range(CPD):
            stage_v[r, pl.ds(cc * LANES, LANES)] = jnp.zeros((LANES,), jnp.float32)
        return 0
    lax.fori_loop(0, STAGE_ROWS, _zero_row, 0)

    row_base = s * ROWS_PER_TEC
    for k in range(ROWS_PER_TEC // STAGE_ROWS):
        pltpu.sync_copy(stage_v, y_sh.at[pl.ds(row_base + k * STAGE_ROWS, STAGE_ROWS)])
    plsc.subcore_barrier()

    row0 = wid * TPW
    pltpu.sync_copy(idxi_hbm.at[pl.ds(row0, TPW)], idxi_v)
    pltpu.sync_copy(idxj_hbm.at[pl.ds(row0, TPW)], idxj_v)

    def _edge_tile(j, _):
        gat = pltpu.async_copy(x_hbm.at[idxj_v.at[j]], xr_v, sem)
        pltpu.sync_copy(wij_hbm.at[pl.ds(wid * EPW + j * B_EDGE, B_EDGE)], wij_v)
        gat.wait()

        def _mul_row(r, _):
            for cc in range(CPD):
                sl = pl.ds(cc * LANES, LANES)
                xr_v[r, sl] = xr_v[r, sl] * wij_v[r, sl]
            return 0
        lax.fori_loop(0, B_EDGE, _mul_row, 0)

        pltpu.sync_copy(xr_v, y_sh.at[idxi_v.at[j]], add=True)
        return 0
    lax.fori_loop(0, TPW, _edge_tile, 0)
    plsc.subcore_barrier()

    for k in range(ROWS_PER_TEC // STAGE_ROWS):
        r0 = row_base + k * STAGE_ROWS
        pltpu.sync_copy(y_sh.at[pl.ds(r0, STAGE_ROWS)], stage_v)
        pltpu.sync_copy(stage_v, out_hbm.at[c].at[pl.ds(r0, STAGE_ROWS)])


_sc_call = pl.kernel(
    _sc_body,
    out_type=jax.ShapeDtypeStruct((NC, N_NODES, D_FEAT), jnp.float32),
    mesh=plsc.VectorSubcoreMesh(core_axis_name="c", subcore_axis_name="s"),
    scratch_types=[
        pltpu.VMEM((TPW, B_EDGE), jnp.int32),
        pltpu.VMEM((TPW, B_EDGE), jnp.int32),
        pltpu.VMEM((B_EDGE, D_FEAT), jnp.float32),
        pltpu.VMEM((B_EDGE, D_FEAT), jnp.float32),
        pltpu.VMEM((STAGE_ROWS, D_FEAT), jnp.float32),
        pltpu.VMEM_SHARED((N_NODES, D_FEAT), jnp.float32),
        pltpu.SemaphoreType.DMA,
    ],
)


def _add_body(a_ref, b_ref, o_ref):
    o_ref[...] = a_ref[0] + b_ref[0]


_ROWS_PER_BLK = 1250


def _combine(partial):
    return pl.pallas_call(
        _add_body,
        out_shape=jax.ShapeDtypeStruct((N_NODES, D_FEAT), jnp.float32),
        grid=(N_NODES // _ROWS_PER_BLK,),
        in_specs=[
            pl.BlockSpec((1, _ROWS_PER_BLK, D_FEAT), lambda i: (0, i, 0)),
            pl.BlockSpec((1, _ROWS_PER_BLK, D_FEAT), lambda i: (1, i, 0)),
        ],
        out_specs=pl.BlockSpec((_ROWS_PER_BLK, D_FEAT), lambda i: (i, 0)),
    )(partial, partial)


def kernel(x, Wij, idx_i, idx_j):
    idx_i2 = idx_i.astype(jnp.int32).reshape(NW * TPW, B_EDGE)
    idx_j2 = idx_j.astype(jnp.int32).reshape(NW * TPW, B_EDGE)
    partial = _sc_call(x, Wij, idx_i2, idx_j2)
    return _combine(partial)


# speedup vs baseline: 4.5309x; 4.5309x over previous
import jax
import jax.numpy as jnp
from jax import lax
from jax.experimental import pallas as pl
from jax.experimental.pallas import tpu as pltpu
from jax.experimental.pallas import tpu_sc as plsc

N_NODES = 10000
N_EDGES = 320000
D_FEAT = 128

NC = 2
NS = 16
B_EDGE = 128
T_TILES = N_EDGES // B_EDGE
HALF = 5120
Y_ROWS = 5248
ZPT = Y_ROWS // NS
WPT = HALF // NS
LANES = 16
CPD = D_FEAT // LANES


def _sc_body(x_hbm, wij_hbm, idxi_hbm, idxj_hbm, t0_hbm, out_hbm,
             idxi_v, idxj_v, idxw_v, wij_v, xr_v, t0_v, y_sh, sem):
    c = lax.axis_index("c")
    s = lax.axis_index("s")

    def _zero_row(r, _):
        for cc in range(CPD):
            wij_v[r, pl.ds(cc * LANES, LANES)] = jnp.zeros((LANES,), jnp.float32)
        return 0
    lax.fori_loop(0, B_EDGE, _zero_row, 0)

    zbase = s * ZPT
    pltpu.sync_copy(wij_v, y_sh.at[pl.ds(zbase, 128)])
    pltpu.sync_copy(wij_v, y_sh.at[pl.ds(zbase + 128, 128)])
    pltpu.sync_copy(wij_v.at[pl.ds(0, ZPT - 256)], y_sh.at[pl.ds(zbase + 256, ZPT - 256)])
    plsc.subcore_barrier()

    pltpu.sync_copy(t0_hbm, t0_v)
    t0 = t0_v[0, pl.ds(0, LANES)][0]
    n_tiles = jnp.where(c == 0, t0 + 1, T_TILES - t0)
    count = (n_tiles - s + (NS - 1)) // NS
    tbase = c * t0 + s
    row_lo = c * HALF

    def _edge_tile(k, _):
        t = tbase + k * NS
        pltpu.sync_copy(idxj_hbm.at[t], idxj_v)
        gat = pltpu.async_copy(x_hbm.at[idxj_v.at[0]], xr_v, sem)
        pltpu.sync_copy(idxi_hbm.at[t], idxi_v)
        pltpu.sync_copy(wij_hbm.at[t], wij_v)

        for cc in range(CPD):
            sl = pl.ds(cc * LANES, LANES)
            local = idxi_v[0, sl] - row_lo
            keep = (local >= 0) & (local < HALF)
            idxw_v[0, sl] = jnp.where(keep, local, HALF)

        gat.wait()

        def _mul_row(r, _):
            for cc in range(CPD):
                sl = pl.ds(cc * LANES, LANES)
                xr_v[r, sl] = xr_v[r, sl] * wij_v[r, sl]
            return 0
        lax.fori_loop(0, B_EDGE, _mul_row, 0)

        pltpu.sync_copy(xr_v, y_sh.at[idxw_v.at[0]], add=True)
        return 0
    lax.fori_loop(0, count, _edge_tile, 0)
    plsc.subcore_barrier()

    wbase = s * WPT
    pltpu.sync_copy(y_sh.at[pl.ds(wbase, WPT)],
                    out_hbm.at[pl.ds(row_lo + wbase, WPT)])


_sc_call = pl.kernel(
    _sc_body,
    out_type=jax.ShapeDtypeStruct((NC * HALF, D_FEAT), jnp.float32),
    mesh=plsc.VectorSubcoreMesh(core_axis_name="c", subcore_axis_name="s"),
    scratch_types=[
        pltpu.VMEM((1, B_EDGE), jnp.int32),
        pltpu.VMEM((1, B_EDGE), jnp.int32),
        pltpu.VMEM((1, B_EDGE), jnp.int32),
        pltpu.VMEM((B_EDGE, D_FEAT), jnp.float32),
        pltpu.VMEM((B_EDGE, D_FEAT), jnp.float32),
        pltpu.VMEM((1, LANES), jnp.int32),
        pltpu.VMEM_SHARED((Y_ROWS, D_FEAT), jnp.float32),
        pltpu.SemaphoreType.DMA,
    ],
)


def kernel(x, Wij, idx_i, idx_j):
    idx_i = idx_i.astype(jnp.int32)
    idx_i2 = idx_i.reshape(T_TILES, 1, B_EDGE)
    idx_j2 = idx_j.astype(jnp.int32).reshape(T_TILES, 1, B_EDGE)
    wij3 = Wij.reshape(T_TILES, B_EDGE, D_FEAT)
    split = jnp.searchsorted(idx_i, HALF).astype(jnp.int32)
    t0 = jnp.full((1, LANES), jnp.minimum(split // B_EDGE, T_TILES - 1),
                  dtype=jnp.int32)
    y = _sc_call(x, wij3, idx_i2, idx_j2, t0)
    return y[:N_NODES]


# speedup vs baseline: 7.1250x; 1.5725x over previous
import jax
import jax.numpy as jnp
from jax import lax
from jax.experimental import pallas as pl
from jax.experimental.pallas import tpu as pltpu
from jax.experimental.pallas import tpu_sc as plsc

N_NODES = 10000
N_EDGES = 320000
D_FEAT = 128

NC = 2
NS = 16
B_EDGE = 128
T_TILES = N_EDGES // B_EDGE
HALF = 5120
Y_ROWS = 5248
ZPT = Y_ROWS // NS
WPT = HALF // NS
LANES = 16
CPD = D_FEAT // LANES


def _sc_body(x_hbm, wij_hbm, idx_hbm, t0_hbm, out_hbm,
             idx_v, idxw_v, wij_v, xr_v, t0_v, y_sh,
             in0, in1, si0, si1, si2, si3):
    c = lax.axis_index("c")
    s = lax.axis_index("s")
    in_sems = (in0, in1)
    idx_sems = (si0, si1, si2, si3)

    def _zero_row(r, _):
        for cc in range(CPD):
            wij_v[0, r, pl.ds(cc * LANES, LANES)] = jnp.zeros((LANES,), jnp.float32)
        return 0
    lax.fori_loop(0, B_EDGE, _zero_row, 0)

    zbase = s * ZPT
    pltpu.sync_copy(wij_v.at[0], y_sh.at[pl.ds(zbase, 128)])
    pltpu.sync_copy(wij_v.at[0], y_sh.at[pl.ds(zbase + 128, 128)])
    pltpu.sync_copy(wij_v.at[0, pl.ds(0, ZPT - 256)],
                    y_sh.at[pl.ds(zbase + 256, ZPT - 256)])
    plsc.subcore_barrier()

    pltpu.sync_copy(t0_hbm, t0_v)
    t0 = t0_v[0, pl.ds(0, LANES)][0]
    n_tiles = jnp.where(c == 0, t0 + 1, T_TILES - t0)
    count = (n_tiles - s + (NS - 1)) // NS
    tbase = c * t0 + s
    row_lo = c * HALF

    def tile_at(m):
        return jnp.minimum(tbase + m * NS, T_TILES - 1)

    def issue_inputs(m, slot, b):
        pltpu.async_copy(x_hbm.at[idx_v.at[slot, 1]], xr_v.at[b], in_sems[b])
        pltpu.async_copy(wij_hbm.at[tile_at(m)], wij_v.at[b], in_sems[b])

    def wait_inputs(b):
        pltpu.make_async_copy(wij_hbm.at[0], xr_v.at[b], in_sems[b]).wait()
        pltpu.make_async_copy(wij_hbm.at[0], wij_v.at[b], in_sems[b]).wait()

    def issue_idx(m, slot):
        pltpu.async_copy(idx_hbm.at[tile_at(m)], idx_v.at[slot], idx_sems[slot])

    def wait_idx(slot):
        pltpu.make_async_copy(idx_hbm.at[0], idx_v.at[slot], idx_sems[slot]).wait()

    pltpu.sync_copy(idx_hbm.at[tile_at(0)], idx_v.at[0])
    pltpu.sync_copy(idx_hbm.at[tile_at(1)], idx_v.at[1])
    issue_idx(2, 2)
    issue_idx(3, 3)
    issue_inputs(0, 0, 0)
    issue_inputs(1, 1, 1)

    def _edge_quad(q, _):
        for i in range(4):
            k = q * 4 + i
            b = i % 2
            slot = i
            wait_inputs(b)

            for cc in range(CPD):
                sl = pl.ds(cc * LANES, LANES)
                local = idx_v[slot, 0, sl] - row_lo
                keep = (local >= 0) & (local < HALF)
                idxw_v[b, 0, sl] = jnp.where(keep, local, HALF)

            issue_idx(k + 4, slot)

            def _mul_row(r, _):
                for cc in range(CPD):
                    sl = pl.ds(cc * LANES, LANES)
                    xr_v[b, r, sl] = xr_v[b, r, sl] * wij_v[b, r, sl]
                return 0
            lax.fori_loop(0, B_EDGE, _mul_row, 0)

            @pl.when(k < count)
            def _():
                pltpu.sync_copy(xr_v.at[b], y_sh.at[idxw_v.at[b, 0]], add=True)

            wait_idx((i + 2) % 4)
            issue_inputs(k + 2, (i + 2) % 4, b)
        return 0
    lax.fori_loop(0, (count + 3) // 4, _edge_quad, 0)

    wait_inputs(0)
    wait_inputs(1)
    wait_idx(2)
    wait_idx(3)

    plsc.subcore_barrier()

    wbase = s * WPT
    pltpu.sync_copy(y_sh.at[pl.ds(wbase, WPT)],
                    out_hbm.at[pl.ds(row_lo + wbase, WPT)])


_sc_call = pl.kernel(
    _sc_body,
    out_type=jax.ShapeDtypeStruct((NC * HALF, D_FEAT), jnp.float32),
    mesh=plsc.VectorSubcoreMesh(core_axis_name="c", subcore_axis_name="s"),
    scratch_types=[
        pltpu.VMEM((4, 2, B_EDGE), jnp.int32),
        pltpu.VMEM((2, 1, B_EDGE), jnp.int32),
        pltpu.VMEM((2, B_EDGE, D_FEAT), jnp.float32),
        pltpu.VMEM((2, B_EDGE, D_FEAT), jnp.float32),
        pltpu.VMEM((1, LANES), jnp.int32),
        pltpu.VMEM_SHARED((Y_ROWS, D_FEAT), jnp.float32),
        pltpu.SemaphoreType.DMA,
        pltpu.SemaphoreType.DMA,
        pltpu.SemaphoreType.DMA,
        pltpu.SemaphoreType.DMA,
        pltpu.SemaphoreType.DMA,
        pltpu.SemaphoreType.DMA,
    ],
)


def kernel(x, Wij, idx_i, idx_j):
    idx_i = idx_i.astype(jnp.int32)
    idx2 = jnp.concatenate(
        [idx_i.reshape(T_TILES, 1, B_EDGE),
         idx_j.astype(jnp.int32).reshape(T_TILES, 1, B_EDGE)], axis=1)
    wij3 = Wij.reshape(T_TILES, B_EDGE, D_FEAT)
    split = jnp.searchsorted(idx_i, HALF).astype(jnp.int32)
    t0 = jnp.full((1, LANES), jnp.minimum(split // B_EDGE, T_TILES - 1),
                  dtype=jnp.int32)
    y = _sc_call(x, wij3, idx2, t0)
    return y[:N_NODES]


# speedup vs baseline: 7.9195x; 1.1115x over previous
import jax
import jax.numpy as jnp
from jax import lax
from jax.experimental import pallas as pl
from jax.experimental.pallas import tpu as pltpu
from jax.experimental.pallas import tpu_sc as plsc

N_NODES = 10000
N_EDGES = 320000
D_FEAT = 128

NC = 2
NS = 16
B_EDGE = 128
T_TILES = N_EDGES // B_EDGE
HALF = 5120
Y_ROWS = 5248
ZPT = Y_ROWS // NS
WPT = HALF // NS
LANES = 16
CPD = D_FEAT // LANES


def _sc_body(x_hbm, wij_hbm, idx_hbm, t0_hbm, out_hbm,
             idx_v, idxw_v, wij_v, xr_v, prod_v, t0_v, y_sh,
             in0, in1, si0, si1, si2, si3, sc_sem):
    c = lax.axis_index("c")
    s = lax.axis_index("s")
    in_sems = (in0, in1)
    idx_sems = (si0, si1, si2, si3)

    def _zero_row(r, _):
        for cc in range(CPD):
            wij_v[0, r, pl.ds(cc * LANES, LANES)] = jnp.zeros((LANES,), jnp.float32)
        return 0
    lax.fori_loop(0, B_EDGE, _zero_row, 0)

    zbase = s * ZPT
    pltpu.sync_copy(wij_v.at[0], y_sh.at[pl.ds(zbase, 128)])
    pltpu.sync_copy(wij_v.at[0], y_sh.at[pl.ds(zbase + 128, 128)])
    pltpu.sync_copy(wij_v.at[0, pl.ds(0, ZPT - 256)],
                    y_sh.at[pl.ds(zbase + 256, ZPT - 256)])
    plsc.subcore_barrier()

    pltpu.sync_copy(t0_hbm, t0_v)
    t0 = t0_v[0, pl.ds(0, LANES)][0]
    n_tiles = jnp.where(c == 0, t0 + 1, T_TILES - t0)
    count = (n_tiles - s + (NS - 1)) // NS
    tbase = c * t0 + s
    row_lo = c * HALF

    def tile_at(m):
        return jnp.minimum(tbase + m * NS, T_TILES - 1)

    def issue_inputs(m, slot, b):
        pltpu.async_copy(x_hbm.at[idx_v.at[slot, 1]], xr_v.at[b], in_sems[b])
        pltpu.async_copy(wij_hbm.at[tile_at(m)], wij_v.at[b], in_sems[b])

    def wait_inputs(b):
        pltpu.make_async_copy(wij_hbm.at[0], xr_v.at[b], in_sems[b]).wait()
        pltpu.make_async_copy(wij_hbm.at[0], wij_v.at[b], in_sems[b]).wait()

    def issue_idx(m, slot):
        pltpu.async_copy(idx_hbm.at[tile_at(m)], idx_v.at[slot], idx_sems[slot])

    def wait_idx(slot):
        pltpu.make_async_copy(idx_hbm.at[0], idx_v.at[slot], idx_sems[slot]).wait()

    pltpu.sync_copy(idx_hbm.at[tile_at(0)], idx_v.at[0])
    pltpu.sync_copy(idx_hbm.at[tile_at(1)], idx_v.at[1])
    issue_idx(2, 2)
    issue_idx(3, 3)
    issue_inputs(0, 0, 0)
    issue_inputs(1, 1, 1)

    def _edge_quad(q, _):
        for i in range(4):
            k = q * 4 + i
            b = i % 2
            slot = i
            wait_inputs(b)

            for cc in range(CPD):
                sl = pl.ds(cc * LANES, LANES)
                local = idx_v[slot, 0, sl] - row_lo
                keep = (local >= 0) & (local < HALF)
                idxw_v[b, 0, sl] = jnp.where(keep, local, HALF)

            issue_idx(k + 4, slot)

            @pl.when((k >= 1) & (k - 1 < count))
            def _():
                pltpu.make_async_copy(
                    prod_v, y_sh.at[idxw_v.at[1 - b, 0]], sc_sem).wait()

            def _mul_row(r, _):
                for cc in range(CPD):
                    sl = pl.ds(cc * LANES, LANES)
                    prod_v[r, sl] = xr_v[b, r, sl] * wij_v[b, r, sl]
                return 0
            lax.fori_loop(0, B_EDGE, _mul_row, 0)

            @pl.when(k < count)
            def _():
                pltpu.async_copy(prod_v, y_sh.at[idxw_v.at[b, 0]], sc_sem,
                                 add=True)

            wait_idx((i + 2) % 4)
            issue_inputs(k + 2, (i + 2) % 4, b)
        return 0
    lax.fori_loop(0, (count + 3) // 4, _edge_quad, 0)

    @pl.when((count > 0) & (count % 4 == 0))
    def _():
        pltpu.make_async_copy(
            prod_v, y_sh.at[idxw_v.at[(count - 1) % 2, 0]], sc_sem).wait()
    wait_inputs(0)
    wait_inputs(1)
    wait_idx(2)
    wait_idx(3)

    plsc.subcore_barrier()

    wbase = s * WPT
    pltpu.sync_copy(y_sh.at[pl.ds(wbase, WPT)],
                    out_hbm.at[pl.ds(row_lo + wbase, WPT)])


_sc_call = pl.kernel(
    _sc_body,
    out_type=jax.ShapeDtypeStruct((NC * HALF, D_FEAT), jnp.float32),
    mesh=plsc.VectorSubcoreMesh(core_axis_name="c", subcore_axis_name="s"),
    scratch_types=[
        pltpu.VMEM((4, 2, B_EDGE), jnp.int32),
        pltpu.VMEM((2, 1, B_EDGE), jnp.int32),
        pltpu.VMEM((2, B_EDGE, D_FEAT), jnp.float32),
        pltpu.VMEM((2, B_EDGE, D_FEAT), jnp.float32),
        pltpu.VMEM((B_EDGE, D_FEAT), jnp.float32),
        pltpu.VMEM((1, LANES), jnp.int32),
        pltpu.VMEM_SHARED((Y_ROWS, D_FEAT), jnp.float32),
        pltpu.SemaphoreType.DMA,
        pltpu.SemaphoreType.DMA,
        pltpu.SemaphoreType.DMA,
        pltpu.SemaphoreType.DMA,
        pltpu.SemaphoreType.DMA,
        pltpu.SemaphoreType.DMA,
        pltpu.SemaphoreType.DMA,
    ],
)


def kernel(x, Wij, idx_i, idx_j):
    idx_i = idx_i.astype(jnp.int32)
    idx2 = jnp.concatenate(
        [idx_i.reshape(T_TILES, 1, B_EDGE),
         idx_j.astype(jnp.int32).reshape(T_TILES, 1, B_EDGE)], axis=1)
    wij3 = Wij.reshape(T_TILES, B_EDGE, D_FEAT)
    split = jnp.searchsorted(idx_i, HALF).astype(jnp.int32)
    t0 = jnp.full((1, LANES), jnp.minimum(split // B_EDGE, T_TILES - 1),
                  dtype=jnp.int32)
    y = _sc_call(x, wij3, idx2, t0)
    return y[:N_NODES]
